# baseline (device time: 208922 ns/iter reference)
import jax
import jax.numpy as jnp
from jax import lax
from jax.experimental import pallas as pl
from jax.experimental.pallas import tpu as pltpu

K = 16


def kernel(x):
    m, n = x.shape
    c = m // K

    def body(x_hbm, out_hbm, vin, vout, load_sems, copy_sems, send_sems, recv_sems):
        my_x = lax.axis_index("x")
        my_y = lax.axis_index("y")
        my_z = lax.axis_index("z")
        buddy = (1 - my_x, my_y, my_z)

        barrier_sem = pltpu.get_barrier_semaphore()
        pl.semaphore_signal(
            barrier_sem, inc=1, device_id=buddy,
            device_id_type=pl.DeviceIdType.MESH,
        )
        pl.semaphore_wait(barrier_sem, 1)

        base = my_x * m

        loads = [None] * K
        copies = [None] * K
        rdmas = [None] * K

        def start_load(i):
            loads[i] = pltpu.make_async_copy(
                x_hbm.at[pl.ds(i * c, c), :], vin.at[i % 2], load_sems.at[i % 2]
            )
            loads[i].start()

        start_load(0)
        for i in range(K):
            s = i % 2
            if i + 1 < K:
                start_load(i + 1)
            loads[i].wait()
            vout[s, :, :] = vin[s, :, :].astype(jnp.bfloat16)
            copies[i] = pltpu.make_async_copy(
                vout.at[s], out_hbm.at[pl.ds(base + i * c, c), :], copy_sems.at[i]
            )
            copies[i].start()
            copies[i].wait()
            rdmas[i] = pltpu.make_async_remote_copy(
                src_ref=out_hbm.at[pl.ds(base + i * c, c), :],
                dst_ref=out_hbm.at[pl.ds(base + i * c, c), :],
                send_sem=send_sems.at[i],
                recv_sem=recv_sems.at[i],
                device_id=buddy,
                device_id_type=pl.DeviceIdType.MESH,
            )
            rdmas[i].start()

        for i in range(K):
            rdmas[i].wait_send()
        for i in range(K):
            rdmas[i].wait_recv()

    return pl.pallas_call(
        body,
        out_shape=jax.ShapeDtypeStruct((2 * m, n), jnp.bfloat16),
        in_specs=[pl.BlockSpec(memory_space=pl.ANY)],
        out_specs=pl.BlockSpec(memory_space=pl.ANY),
        scratch_shapes=[
            pltpu.VMEM((2, c, n), jnp.float32),
            pltpu.VMEM((2, c, n), jnp.bfloat16),
            pltpu.SemaphoreType.DMA((2,)),
            pltpu.SemaphoreType.DMA((K,)),
            pltpu.SemaphoreType.DMA((K,)),
            pltpu.SemaphoreType.DMA((K,)),
        ],
        compiler_params=pltpu.CompilerParams(collective_id=0),
    )(x)


# device time: 120948 ns/iter; 1.7274x vs baseline; 1.7274x over previous
import jax
import jax.numpy as jnp
from jax import lax
from jax.experimental import pallas as pl
from jax.experimental.pallas import tpu as pltpu

NC = 32
E_N = 4
QN = 7
SPLIT = 4
NX = E_N + QN
NY = QN + SPLIT
NZ = QN + (QN - SPLIT)


def kernel(x):
    m, n = x.shape
    cr = m // NC

    def body(x_hbm, out_hbm, vin, vout, xbuf,
             load_sems, copy_sems, xs, xr, ys, yr, zs, zr):
        my_x = lax.axis_index("x")
        my_y = lax.axis_index("y")
        my_z = lax.axis_index("z")
        a = lax.rem(my_y, 2)
        b = lax.rem(my_z, 2)
        xb = (1 - my_x, my_y, my_z)
        yb = (my_x, my_y + 1 - 2 * a, my_z)
        zb = (my_x, my_y, my_z + 1 - 2 * b)

        barrier_sem = pltpu.get_barrier_semaphore()
        for nbr in (xb, yb, zb):
            pl.semaphore_signal(
                barrier_sem, inc=1, device_id=nbr,
                device_id_type=pl.DeviceIdType.MESH,
            )
        pl.semaphore_wait(barrier_sem, 3)

        own = my_x * m
        fb = (1 - my_x) * m

        qsel = 2 * a + b

        def qbase(k):
            return E_N + lax.rem(qsel + k, 4) * QN

        def ord_idx(i):
            if i < QN:
                return qbase(0) + i
            if i < NX:
                return i - QN
            k = 1 + (i - NX) // QN
            return qbase(k) + (i - NX) % QN

        loads = [None] * NC
        copies = [None] * NC
        xsend = [None] * NX

        def start_load(i):
            g = ord_idx(i)
            loads[i] = pltpu.make_async_copy(
                x_hbm.at[pl.ds(g * cr, cr), :], vin.at[i % 4],
                load_sems.at[i % 4],
            )
            loads[i].start()

        def cast_step(i):
            g = ord_idx(i)
            if i + 3 < NC:
                start_load(i + 3)
            loads[i].wait()
            if i < NX:
                xbuf[i, :, :] = vin[i % 4, :, :].astype(jnp.bfloat16)
                copies[i] = pltpu.make_async_copy(
                    xbuf.at[i], out_hbm.at[pl.ds(own + g * cr, cr), :],
                    copy_sems.at[i],
                )
                copies[i].start()
                xsend[i] = pltpu.make_async_remote_copy(
                    src_ref=xbuf.at[i],
                    dst_ref=out_hbm.at[pl.ds(own + g * cr, cr), :],
                    send_sem=xs.at[i],
                    recv_sem=xr.at[i],
                    device_id=xb,
                    device_id_type=pl.DeviceIdType.MESH,
                )
                xsend[i].start()
            else:
                s = i % 4
                if i - 4 >= NX:
                    copies[i - 4].wait()
                vout[s, :, :] = vin[i % 4, :, :].astype(jnp.bfloat16)
                copies[i] = pltpu.make_async_copy(
                    vout.at[s], out_hbm.at[pl.ds(own + g * cr, cr), :],
                    copy_sems.at[i],
                )
                copies[i].start()

        for i in range(3):
            start_load(i)
        for i in range(NX):
            cast_step(i)

        def xin_idx(j):
            return qbase(0) + j if j < QN else j - QN

        def yin_idx(j):
            yq = E_N + (2 * (1 - a) + b) * QN
            dq = E_N + (2 * (1 - a) + (1 - b)) * QN
            return yq + j if j < QN else dq + (j - QN)

        def zin_idx(j):
            zq = E_N + (2 * a + (1 - b)) * QN
            dq = E_N + (2 * (1 - a) + (1 - b)) * QN
            return zq + j if j < QN else dq + SPLIT + (j - QN)

        def recv_desc(rows, sem):
            return pltpu.make_async_remote_copy(
                src_ref=out_hbm.at[rows],
                dst_ref=out_hbm.at[rows],
                send_sem=xs.at[0],
                recv_sem=sem,
                device_id=xb,
                device_id_type=pl.DeviceIdType.MESH,
            )

        def fwd(g, send_sems, recv_sems, slot, dev):
            rows = pl.ds(fb + g * cr, cr)
            r = pltpu.make_async_remote_copy(
                src_ref=out_hbm.at[rows],
                dst_ref=out_hbm.at[rows],
                send_sem=send_sems.at[slot],
                recv_sem=recv_sems.at[slot],
                device_id=dev,
                device_id_type=pl.DeviceIdType.MESH,
            )
            r.start()
            return r

        xrecv = [recv_desc(pl.ds(fb + xin_idx(j) * cr, cr), xr.at[j])
                 for j in range(NX)]
        yrecv = [recv_desc(pl.ds(fb + yin_idx(j) * cr, cr), yr.at[j])
                 for j in range(NY)]
        zrecv = [recv_desc(pl.ds(fb + zin_idx(j) * cr, cr), zr.at[j])
                 for j in range(NZ)]

        ysend = [None] * NY
        zsend = [None] * NZ
        ci = NX

        for j in range(QN):
            for _ in range(3):
                if ci < NC:
                    cast_step(ci)
                    ci += 1
            xrecv[j].wait_recv()
            g = qbase(0) + j
            ysend[j] = fwd(g, ys, yr, j, yb)
            zsend[j] = fwd(g, zs, zr, j, zb)
            if j < SPLIT:
                zrecv[j].wait_recv()
                ysend[QN + j] = fwd(zin_idx(j), ys, yr, QN + j, yb)
            else:
                yrecv[j].wait_recv()
                zsend[QN + j - SPLIT] = fwd(
                    yin_idx(j), zs, zr, QN + j - SPLIT, zb)
        while ci < NC:
            cast_step(ci)
            ci += 1

        for j in range(QN, NX):
            xrecv[j].wait_recv()
        for j in range(SPLIT):
            yrecv[j].wait_recv()
        for j in range(QN, NY):
            yrecv[j].wait_recv()
        for j in range(SPLIT, QN):
            zrecv[j].wait_recv()
        for j in range(QN, NZ):
            zrecv[j].wait_recv()
        for i in range(NX):
            copies[i].wait()
        for i in range(max(NX, NC - 4), NC):
            copies[i].wait()
        for i in range(NX):
            xsend[i].wait_send()
        for r in ysend:
            r.wait_send()
        for r in zsend:
            r.wait_send()

    return pl.pallas_call(
        body,
        out_shape=jax.ShapeDtypeStruct((2 * m, n), jnp.bfloat16),
        in_specs=[pl.BlockSpec(memory_space=pl.ANY)],
        out_specs=pl.BlockSpec(memory_space=pl.ANY),
        scratch_shapes=[
            pltpu.VMEM((4, cr, n), jnp.float32),
            pltpu.VMEM((4, cr, n), jnp.bfloat16),
            pltpu.VMEM((NX, cr, n), jnp.bfloat16),
            pltpu.SemaphoreType.DMA((4,)),
            pltpu.SemaphoreType.DMA((NC,)),
            pltpu.SemaphoreType.DMA((NX,)),
            pltpu.SemaphoreType.DMA((NX,)),
            pltpu.SemaphoreType.DMA((NY,)),
            pltpu.SemaphoreType.DMA((NY,)),
            pltpu.SemaphoreType.DMA((NZ,)),
            pltpu.SemaphoreType.DMA((NZ,)),
        ],
        compiler_params=pltpu.CompilerParams(collective_id=0),
    )(x)


# device time: 119610 ns/iter; 1.7467x vs baseline; 1.0112x over previous
import jax
import jax.numpy as jnp
from jax import lax
from jax.experimental import pallas as pl
from jax.experimental.pallas import tpu as pltpu

NC = 32
E_N = 4
QN = 7
SPLIT = 4
NX = E_N + QN
NY = QN + SPLIT
NZ = QN + (QN - SPLIT)


def kernel(x):
    m, n = x.shape
    cr = m // NC

    def body(x_hbm, out_hbm, vin, vout, xbuf, xinb, yinb, zinb,
             load_sems, copy_sems, fxc, fyc, fzc,
             xs, xr, ys, yr, zs, zr):
        my_x = lax.axis_index("x")
        my_y = lax.axis_index("y")
        my_z = lax.axis_index("z")
        a = lax.rem(my_y, 2)
        b = lax.rem(my_z, 2)
        xb = (1 - my_x, my_y, my_z)
        yb = (my_x, my_y + 1 - 2 * a, my_z)
        zb = (my_x, my_y, my_z + 1 - 2 * b)

        barrier_sem = pltpu.get_barrier_semaphore()
        for nbr in (xb, yb, zb):
            pl.semaphore_signal(
                barrier_sem, inc=1, device_id=nbr,
                device_id_type=pl.DeviceIdType.MESH,
            )
        pl.semaphore_wait(barrier_sem, 3)

        own = my_x * m
        fb = (1 - my_x) * m

        qsel = 2 * a + b

        def qbase(k):
            return E_N + lax.rem(qsel + k, 4) * QN

        def ord_idx(i):
            if i < QN:
                return qbase(0) + i
            if i < NX:
                return i - QN
            k = 1 + (i - NX) // QN
            return qbase(k) + (i - NX) % QN

        loads = [None] * NC
        copies = [None] * NC
        xsend = [None] * NX

        def start_load(i):
            g = ord_idx(i)
            loads[i] = pltpu.make_async_copy(
                x_hbm.at[pl.ds(g * cr, cr), :], vin.at[i % 4],
                load_sems.at[i % 4],
            )
            loads[i].start()

        def cast_step(i):
            g = ord_idx(i)
            if i + 3 < NC:
                start_load(i + 3)
            loads[i].wait()
            if i < NX:
                xbuf[i, :, :] = vin[i % 4, :, :].astype(jnp.bfloat16)
                copies[i] = pltpu.make_async_copy(
                    xbuf.at[i], out_hbm.at[pl.ds(own + g * cr, cr), :],
                    copy_sems.at[i],
                )
                copies[i].start()
                xsend[i] = pltpu.make_async_remote_copy(
                    src_ref=xbuf.at[i],
                    dst_ref=xinb.at[i],
                    send_sem=xs.at[i],
                    recv_sem=xr.at[i],
                    device_id=xb,
                    device_id_type=pl.DeviceIdType.MESH,
                )
                xsend[i].start()
            else:
                s = i % 4
                if i - 4 >= NX:
                    copies[i - 4].wait()
                vout[s, :, :] = vin[i % 4, :, :].astype(jnp.bfloat16)
                copies[i] = pltpu.make_async_copy(
                    vout.at[s], out_hbm.at[pl.ds(own + g * cr, cr), :],
                    copy_sems.at[i],
                )
                copies[i].start()

        for i in range(3):
            start_load(i)
        for i in range(NX):
            cast_step(i)

        def xin_idx(j):
            return qbase(0) + j if j < QN else j - QN

        def yin_idx(j):
            yq = E_N + (2 * (1 - a) + b) * QN
            dq = E_N + (2 * (1 - a) + (1 - b)) * QN
            return yq + j if j < QN else dq + (j - QN)

        def zin_idx(j):
            zq = E_N + (2 * a + (1 - b)) * QN
            dq = E_N + (2 * (1 - a) + (1 - b)) * QN
            return zq + j if j < QN else dq + SPLIT + (j - QN)

        def recv_desc(slot_ref, sem):
            return pltpu.make_async_remote_copy(
                src_ref=slot_ref,
                dst_ref=slot_ref,
                send_sem=xs.at[0],
                recv_sem=sem,
                device_id=xb,
                device_id_type=pl.DeviceIdType.MESH,
            )

        xrecv = [recv_desc(xinb.at[j], xr.at[j]) for j in range(NX)]
        yrecv = [recv_desc(yinb.at[j], yr.at[j]) for j in range(NY)]
        zrecv = [recv_desc(zinb.at[j], zr.at[j]) for j in range(NZ)]

        def fwd(src_slot, dst_slot, send_sems, recv_sems, slot, dev):
            r = pltpu.make_async_remote_copy(
                src_ref=src_slot,
                dst_ref=dst_slot,
                send_sem=send_sems.at[slot],
                recv_sem=recv_sems.at[slot],
                device_id=dev,
                device_id_type=pl.DeviceIdType.MESH,
            )
            r.start()
            return r

        fcx = [None] * NX
        fcy = [None] * NY
        fcz = [None] * NZ

        def drain_x(j):
            fcx[j] = pltpu.make_async_copy(
                xinb.at[j], out_hbm.at[pl.ds(fb + xin_idx(j) * cr, cr), :],
                fxc.at[j],
            )
            fcx[j].start()

        def drain_y(j):
            fcy[j] = pltpu.make_async_copy(
                yinb.at[j], out_hbm.at[pl.ds(fb + yin_idx(j) * cr, cr), :],
                fyc.at[j],
            )
            fcy[j].start()

        def drain_z(j):
            fcz[j] = pltpu.make_async_copy(
                zinb.at[j], out_hbm.at[pl.ds(fb + zin_idx(j) * cr, cr), :],
                fzc.at[j],
            )
            fcz[j].start()

        ysend = [None] * NY
        zsend = [None] * NZ
        ci = NX

        for j in range(QN):
            for _ in range(3):
                if ci < NC:
                    cast_step(ci)
                    ci += 1
            xrecv[j].wait_recv()
            drain_x(j)
            ysend[j] = fwd(xinb.at[j], yinb.at[j], ys, yr, j, yb)
            zsend[j] = fwd(xinb.at[j], zinb.at[j], zs, zr, j, zb)
            if j < SPLIT:
                zrecv[j].wait_recv()
                drain_z(j)
                ysend[QN + j] = fwd(
                    zinb.at[j], yinb.at[QN + j], ys, yr, QN + j, yb)
            else:
                yrecv[j].wait_recv()
                drain_y(j)
                zsend[QN + j - SPLIT] = fwd(
                    yinb.at[j], zinb.at[QN + j - SPLIT],
                    zs, zr, QN + j - SPLIT, zb)
        while ci < NC:
            cast_step(ci)
            ci += 1

        for j in range(QN, NX):
            xrecv[j].wait_recv()
            drain_x(j)
        for j in range(SPLIT):
            yrecv[j].wait_recv()
            drain_y(j)
        for j in range(QN, NY):
            yrecv[j].wait_recv()
            drain_y(j)
        for j in range(SPLIT, QN):
            zrecv[j].wait_recv()
            drain_z(j)
        for j in range(QN, NZ):
            zrecv[j].wait_recv()
            drain_z(j)
        for i in range(NX):
            copies[i].wait()
        for i in range(max(NX, NC - 4), NC):
            copies[i].wait()
        for i in range(NX):
            xsend[i].wait_send()
        for r in ysend:
            r.wait_send()
        for r in zsend:
            r.wait_send()
        for c in fcx:
            c.wait()
        for c in fcy:
            c.wait()
        for c in fcz:
            c.wait()

    return pl.pallas_call(
        body,
        out_shape=jax.ShapeDtypeStruct((2 * m, n), jnp.bfloat16),
        in_specs=[pl.BlockSpec(memory_space=pl.ANY)],
        out_specs=pl.BlockSpec(memory_space=pl.ANY),
        scratch_shapes=[
            pltpu.VMEM((4, cr, n), jnp.float32),
            pltpu.VMEM((4, cr, n), jnp.bfloat16),
            pltpu.VMEM((NX, cr, n), jnp.bfloat16),
            pltpu.VMEM((NX, cr, n), jnp.bfloat16),
            pltpu.VMEM((NY, cr, n), jnp.bfloat16),
            pltpu.VMEM((NZ, cr, n), jnp.bfloat16),
            pltpu.SemaphoreType.DMA((4,)),
            pltpu.SemaphoreType.DMA((NC,)),
            pltpu.SemaphoreType.DMA((NX,)),
            pltpu.SemaphoreType.DMA((NY,)),
            pltpu.SemaphoreType.DMA((NZ,)),
            pltpu.SemaphoreType.DMA((NX,)),
            pltpu.SemaphoreType.DMA((NX,)),
            pltpu.SemaphoreType.DMA((NY,)),
            pltpu.SemaphoreType.DMA((NY,)),
            pltpu.SemaphoreType.DMA((NZ,)),
            pltpu.SemaphoreType.DMA((NZ,)),
        ],
        compiler_params=pltpu.CompilerParams(collective_id=0),
    )(x)
